# baseline (speedup 1.0000x reference)
import functools

import jax
import jax.numpy as jnp
from jax import lax
from jax.experimental import pallas as pl
from jax.experimental.pallas import tpu as pltpu
from jax.experimental.pallas import tpu_sc as plsc

BS, C, NUM_IMG, H, W = 32, 128, 8, 16, 16
P = NUM_IMG * H * W
K = 128
FEAT_DIM, NODE_DIM = 128, 64

_NC, _NS = 2, 16


def _hidden_body(fm_ref, w1_ref, b1_ref, out_ref):
    fm = fm_ref[0]
    h = lax.dot_general(fm, w1_ref[...], (((0,), (0,)), ((), ())),
                        preferred_element_type=jnp.float32)
    out_ref[0] = jnp.maximum(h + b1_ref[...], 0.0)


@functools.lru_cache(maxsize=1)
def _make_gather():
    mesh = plsc.VectorSubcoreMesh(core_axis_name="c", subcore_axis_name="s",
                                  num_cores=_NC, num_subcores=_NS)

    @functools.partial(
        pl.kernel, mesh=mesh,
        out_type=jax.ShapeDtypeStruct((BS * K, 2 * NODE_DIM), jnp.float32),
        scratch_types=[
            pltpu.VMEM((K,), jnp.float32),
            pltpu.VMEM((K,), jnp.float32),
            pltpu.VMEM((K,), jnp.float32),
            pltpu.VMEM((K,), jnp.int32),
            pltpu.VMEM((K, 2 * NODE_DIM), jnp.float32),
            pltpu.SemaphoreType.DMA,
        ],
    )
    def gather_k(pp_hbm, table_hbm, out_hbm, v_v, r_v, c_v, idx_v, rows_v,
                 sem):
        w = lax.axis_index("s") * _NC + lax.axis_index("c")
        base = w * K
        pltpu.sync_copy(pp_hbm.at[pl.ds(base, K)], v_v)
        pltpu.sync_copy(pp_hbm.at[pl.ds(BS * K + base, K)], r_v)
        pltpu.sync_copy(pp_hbm.at[pl.ds(2 * BS * K + base, K)], c_v)
        for i in range(K // 16):
            s = pl.ds(i * 16, 16)
            vi = jnp.clip(v_v[s], 0.0, float(NUM_IMG - 1)).astype(jnp.int32)
            ri = jnp.clip(r_v[s], 0.0, float(H - 1)).astype(jnp.int32)
            ci = jnp.clip(c_v[s], 0.0, float(W - 1)).astype(jnp.int32)
            t = w * P + vi * (H * W) + ri * W + ci
            idx_v[s] = lax.shift_right_logical(t, 1)
        pltpu.async_copy(table_hbm.at[idx_v], rows_v, sem).wait()
        pltpu.sync_copy(rows_v, out_hbm.at[pl.ds(base, K)])

    return gather_k


def _out_body(g_ref, ccol_ref, w2s_ref, b2_ref, out_ref):
    ci = jnp.clip(ccol_ref[...].astype(jnp.int32), 0, W - 1)
    parity = lax.rem(ci, 2)
    g = g_ref[0]
    col = lax.broadcasted_iota(jnp.int32, (K, 2 * NODE_DIM), 1)
    half = jnp.where(col < NODE_DIM, 0, 1)
    gm = jnp.where(half == parity, g, 0.0)
    w2s = jnp.concatenate([w2s_ref[...], w2s_ref[...]], axis=0)
    emb = lax.dot_general(gm, w2s, (((1,), (0,)), ((), ())),
                          preferred_element_type=jnp.float32) + b2_ref[...]
    norm = jnp.maximum(jnp.sqrt(jnp.sum(emb * emb, axis=-1, keepdims=True)),
                       1e-12)
    out_ref[0] = emb / norm


def kernel(feature_map, peak_positions, W1, b1, W2, b2):
    fm = feature_map.reshape(BS, C, P)
    b1r = b1.reshape(1, NODE_DIM)
    b2r = b2.reshape(1, NODE_DIM)

    table = pl.pallas_call(
        _hidden_body,
        grid=(BS,),
        in_specs=[
            pl.BlockSpec((1, C, P), lambda b: (b, 0, 0)),
            pl.BlockSpec((FEAT_DIM, NODE_DIM), lambda b: (0, 0)),
            pl.BlockSpec((1, NODE_DIM), lambda b: (0, 0)),
        ],
        out_specs=pl.BlockSpec((1, P, NODE_DIM), lambda b: (b, 0, 0)),
        out_shape=jax.ShapeDtypeStruct((BS, P, NODE_DIM), jnp.float32),
    )(fm, W1, b1r)

    pp = jnp.transpose(peak_positions, (2, 0, 1)).reshape(3 * BS * K)
    gathered = _make_gather()(pp, table.reshape(BS * P // 2, 2 * NODE_DIM))

    ccol = peak_positions[:, :, 2].reshape(BS * K, 1)
    out = pl.pallas_call(
        _out_body,
        grid=(BS,),
        in_specs=[
            pl.BlockSpec((1, K, 2 * NODE_DIM), lambda b: (b, 0, 0)),
            pl.BlockSpec((K, 1), lambda b: (b, 0)),
            pl.BlockSpec((NODE_DIM, NODE_DIM), lambda b: (0, 0)),
            pl.BlockSpec((1, NODE_DIM), lambda b: (0, 0)),
        ],
        out_specs=pl.BlockSpec((1, K, NODE_DIM), lambda b: (b, 0, 0)),
        out_shape=jax.ShapeDtypeStruct((BS, K, NODE_DIM), jnp.float32),
    )(gathered.reshape(BS, K, 2 * NODE_DIM), ccol, W2, b2r)
    return out


# speedup vs baseline: 1.2816x; 1.2816x over previous
import functools

import jax
import jax.numpy as jnp
from jax import lax
from jax.experimental import pallas as pl
from jax.experimental.pallas import tpu as pltpu
from jax.experimental.pallas import tpu_sc as plsc

BS, C, NUM_IMG, H, W = 32, 128, 8, 16, 16
P = NUM_IMG * H * W
HP = P // 2
K = 128
FEAT_DIM, NODE_DIM = 128, 64

_NC, _NS = 2, 16


def _hidden_body(fm_ref, w1lo_ref, w1hi_ref, b1s_ref, out_ref):
    fm = fm_ref[0]
    fa = fm[:, :HP]
    fb = fm[:, HP:]
    lo = lax.dot_general(fa, w1lo_ref[...], (((0,), (0,)), ((), ())),
                         preferred_element_type=jnp.float32)
    hi = lax.dot_general(fb, w1hi_ref[...], (((0,), (0,)), ((), ())),
                         preferred_element_type=jnp.float32)
    out_ref[0] = jnp.maximum(lo + hi + b1s_ref[...], 0.0)


@functools.lru_cache(maxsize=1)
def _make_gather():
    mesh = plsc.VectorSubcoreMesh(core_axis_name="c", subcore_axis_name="s",
                                  num_cores=_NC, num_subcores=_NS)

    @functools.partial(
        pl.kernel, mesh=mesh,
        out_type=jax.ShapeDtypeStruct((BS * K, 2 * NODE_DIM), jnp.float32),
        scratch_types=[
            pltpu.VMEM((K,), jnp.float32),
            pltpu.VMEM((K,), jnp.float32),
            pltpu.VMEM((K,), jnp.float32),
            pltpu.VMEM((K,), jnp.int32),
            pltpu.VMEM((K, 2 * NODE_DIM), jnp.float32),
            pltpu.SemaphoreType.DMA,
        ],
    )
    def gather_k(pp_hbm, table_hbm, out_hbm, v_v, r_v, c_v, idx_v, rows_v,
                 sem):
        w = lax.axis_index("s") * _NC + lax.axis_index("c")
        base = w * K
        pltpu.sync_copy(pp_hbm.at[pl.ds(base, K)], v_v)
        pltpu.sync_copy(pp_hbm.at[pl.ds(BS * K + base, K)], r_v)
        pltpu.sync_copy(pp_hbm.at[pl.ds(2 * BS * K + base, K)], c_v)
        for i in range(K // 16):
            s = pl.ds(i * 16, 16)
            vi = jnp.clip(v_v[s], 0.0, float(NUM_IMG - 1)).astype(jnp.int32)
            ri = jnp.clip(r_v[s], 0.0, float(H - 1)).astype(jnp.int32)
            ci = jnp.clip(c_v[s], 0.0, float(W - 1)).astype(jnp.int32)
            t = vi * (H * W) + ri * W + ci
            idx_v[s] = w * HP + jnp.bitwise_and(t, HP - 1)
        pltpu.async_copy(table_hbm.at[idx_v], rows_v, sem).wait()
        pltpu.sync_copy(rows_v, out_hbm.at[pl.ds(base, K)])

    return gather_k


def _out_body(g_ref, vcol_ref, w2_ref, b2_ref, out_ref):
    vi = jnp.clip(vcol_ref[...].astype(jnp.int32), 0, NUM_IMG - 1)
    sel = jnp.where(vi >= NUM_IMG // 2, 1, 0)
    g = g_ref[0]
    col = lax.broadcasted_iota(jnp.int32, (K, 2 * NODE_DIM), 1)
    half = jnp.where(col < NODE_DIM, 0, 1)
    gm = jnp.where(half == sel, g, 0.0)
    w2s = jnp.concatenate([w2_ref[...], w2_ref[...]], axis=0)
    emb = lax.dot_general(gm, w2s, (((1,), (0,)), ((), ())),
                          preferred_element_type=jnp.float32) + b2_ref[...]
    norm = jnp.maximum(jnp.sqrt(jnp.sum(emb * emb, axis=-1, keepdims=True)),
                       1e-12)
    out_ref[0] = emb / norm


def kernel(feature_map, peak_positions, W1, b1, W2, b2):
    fm = feature_map.reshape(BS, C, P)
    zeros = jnp.zeros_like(W1)
    w1lo = jnp.concatenate([W1, zeros], axis=1)
    w1hi = jnp.concatenate([zeros, W1], axis=1)
    b1s = jnp.concatenate([b1, b1]).reshape(1, 2 * NODE_DIM)
    b2r = b2.reshape(1, NODE_DIM)

    table = pl.pallas_call(
        _hidden_body,
        grid=(BS,),
        in_specs=[
            pl.BlockSpec((1, C, P), lambda b: (b, 0, 0)),
            pl.BlockSpec((FEAT_DIM, 2 * NODE_DIM), lambda b: (0, 0)),
            pl.BlockSpec((FEAT_DIM, 2 * NODE_DIM), lambda b: (0, 0)),
            pl.BlockSpec((1, 2 * NODE_DIM), lambda b: (0, 0)),
        ],
        out_specs=pl.BlockSpec((1, HP, 2 * NODE_DIM), lambda b: (b, 0, 0)),
        out_shape=jax.ShapeDtypeStruct((BS, HP, 2 * NODE_DIM), jnp.float32),
    )(fm, w1lo, w1hi, b1s)

    pp = jnp.transpose(peak_positions, (2, 0, 1)).reshape(3 * BS * K)
    gathered = _make_gather()(pp, table.reshape(BS * HP, 2 * NODE_DIM))

    vcol = peak_positions[:, :, 0].reshape(BS * K, 1)
    out = pl.pallas_call(
        _out_body,
        grid=(BS,),
        in_specs=[
            pl.BlockSpec((1, K, 2 * NODE_DIM), lambda b: (b, 0, 0)),
            pl.BlockSpec((K, 1), lambda b: (b, 0)),
            pl.BlockSpec((NODE_DIM, NODE_DIM), lambda b: (0, 0)),
            pl.BlockSpec((1, NODE_DIM), lambda b: (0, 0)),
        ],
        out_specs=pl.BlockSpec((1, K, NODE_DIM), lambda b: (b, 0, 0)),
        out_shape=jax.ShapeDtypeStruct((BS, K, NODE_DIM), jnp.float32),
    )(gathered.reshape(BS, K, 2 * NODE_DIM), vcol, W2, b2r)
    return out


# speedup vs baseline: 1.7833x; 1.3914x over previous
import functools

import jax
import jax.numpy as jnp
from jax import lax
from jax.experimental import pallas as pl
from jax.experimental.pallas import tpu as pltpu
from jax.experimental.pallas import tpu_sc as plsc

BS, C, NUM_IMG, H, W = 32, 128, 8, 16, 16
P = NUM_IMG * H * W
HP = P // 2
K = 128
FEAT_DIM, NODE_DIM = 128, 64

_NC, _NS = 2, 16


MBA = 8
MBC = 8


def _hidden_body(fm_ref, w1lo_ref, w1hi_ref, b1s_ref, out_ref):
    for j in range(MBA):
        fm = fm_ref[j]
        fa = fm[:, :HP]
        fb = fm[:, HP:]
        lo = lax.dot_general(fa, w1lo_ref[...], (((0,), (0,)), ((), ())),
                             preferred_element_type=jnp.float32)
        hi = lax.dot_general(fb, w1hi_ref[...], (((0,), (0,)), ((), ())),
                             preferred_element_type=jnp.float32)
        out_ref[j] = jnp.maximum(lo + hi + b1s_ref[...], 0.0)


@functools.lru_cache(maxsize=1)
def _make_gather():
    mesh = plsc.VectorSubcoreMesh(core_axis_name="c", subcore_axis_name="s",
                                  num_cores=_NC, num_subcores=_NS)

    @functools.partial(
        pl.kernel, mesh=mesh,
        out_type=jax.ShapeDtypeStruct((BS * K, 2 * NODE_DIM), jnp.float32),
        scratch_types=[
            pltpu.VMEM((K,), jnp.float32),
            pltpu.VMEM((K,), jnp.float32),
            pltpu.VMEM((K,), jnp.float32),
            pltpu.VMEM((K,), jnp.int32),
            pltpu.VMEM((K, 2 * NODE_DIM), jnp.float32),
            pltpu.SemaphoreType.DMA,
        ],
    )
    def gather_k(pp_hbm, table_hbm, out_hbm, v_v, r_v, c_v, idx_v, rows_v,
                 sem):
        w = lax.axis_index("s") * _NC + lax.axis_index("c")
        base = w * K
        pltpu.sync_copy(pp_hbm.at[pl.ds(base, K)], v_v)
        pltpu.sync_copy(pp_hbm.at[pl.ds(BS * K + base, K)], r_v)
        pltpu.sync_copy(pp_hbm.at[pl.ds(2 * BS * K + base, K)], c_v)
        for i in range(K // 16):
            s = pl.ds(i * 16, 16)
            vi = jnp.clip(v_v[s], 0.0, float(NUM_IMG - 1)).astype(jnp.int32)
            ri = jnp.clip(r_v[s], 0.0, float(H - 1)).astype(jnp.int32)
            ci = jnp.clip(c_v[s], 0.0, float(W - 1)).astype(jnp.int32)
            t = vi * (H * W) + ri * W + ci
            idx_v[s] = w * HP + jnp.bitwise_and(t, HP - 1)
        pltpu.async_copy(table_hbm.at[idx_v], rows_v, sem).wait()
        pltpu.sync_copy(rows_v, out_hbm.at[pl.ds(base, K)])

    return gather_k


def _out_body(g_ref, vcol_ref, w2_ref, b2_ref, out_ref):
    n = MBC * K
    vi = jnp.clip(vcol_ref[...].astype(jnp.int32), 0, NUM_IMG - 1)
    sel = jnp.where(vi >= NUM_IMG // 2, 1, 0)
    g = g_ref[...]
    col = lax.broadcasted_iota(jnp.int32, (n, 2 * NODE_DIM), 1)
    half = jnp.where(col < NODE_DIM, 0, 1)
    gm = jnp.where(half == sel, g, 0.0)
    w2s = jnp.concatenate([w2_ref[...], w2_ref[...]], axis=0)
    emb = lax.dot_general(gm, w2s, (((1,), (0,)), ((), ())),
                          preferred_element_type=jnp.float32) + b2_ref[...]
    norm = jnp.maximum(jnp.sqrt(jnp.sum(emb * emb, axis=-1, keepdims=True)),
                       1e-12)
    out_ref[...] = emb / norm


def kernel(feature_map, peak_positions, W1, b1, W2, b2):
    fm = feature_map.reshape(BS, C, P)
    zeros = jnp.zeros_like(W1)
    w1lo = jnp.concatenate([W1, zeros], axis=1)
    w1hi = jnp.concatenate([zeros, W1], axis=1)
    b1s = jnp.concatenate([b1, b1]).reshape(1, 2 * NODE_DIM)
    b2r = b2.reshape(1, NODE_DIM)

    table = pl.pallas_call(
        _hidden_body,
        grid=(BS // MBA,),
        in_specs=[
            pl.BlockSpec((MBA, C, P), lambda b: (b, 0, 0)),
            pl.BlockSpec((FEAT_DIM, 2 * NODE_DIM), lambda b: (0, 0)),
            pl.BlockSpec((FEAT_DIM, 2 * NODE_DIM), lambda b: (0, 0)),
            pl.BlockSpec((1, 2 * NODE_DIM), lambda b: (0, 0)),
        ],
        out_specs=pl.BlockSpec((MBA, HP, 2 * NODE_DIM), lambda b: (b, 0, 0)),
        out_shape=jax.ShapeDtypeStruct((BS, HP, 2 * NODE_DIM), jnp.float32),
    )(fm, w1lo, w1hi, b1s)

    pp = jnp.transpose(peak_positions, (2, 0, 1)).reshape(3 * BS * K)
    gathered = _make_gather()(pp, table.reshape(BS * HP, 2 * NODE_DIM))

    vcol = peak_positions[:, :, 0].reshape(BS * K, 1)
    out = pl.pallas_call(
        _out_body,
        grid=(BS // MBC,),
        in_specs=[
            pl.BlockSpec((MBC * K, 2 * NODE_DIM), lambda b: (b, 0)),
            pl.BlockSpec((MBC * K, 1), lambda b: (b, 0)),
            pl.BlockSpec((NODE_DIM, NODE_DIM), lambda b: (0, 0)),
            pl.BlockSpec((1, NODE_DIM), lambda b: (0, 0)),
        ],
        out_specs=pl.BlockSpec((MBC * K, NODE_DIM), lambda b: (b, 0)),
        out_shape=jax.ShapeDtypeStruct((BS * K, NODE_DIM), jnp.float32),
    )(gathered, vcol, W2, b2r)
    return out.reshape(BS, K, NODE_DIM)


# speedup vs baseline: 1.8090x; 1.0144x over previous
import functools

import jax
import jax.numpy as jnp
from jax import lax
from jax.experimental import pallas as pl
from jax.experimental.pallas import tpu as pltpu
from jax.experimental.pallas import tpu_sc as plsc

BS, C, NUM_IMG, H, W = 32, 128, 8, 16, 16
P = NUM_IMG * H * W
HP = P // 2
K = 128
FEAT_DIM, NODE_DIM = 128, 64

_NC, _NS = 2, 16


MBA = 8
MBC = 16


def _hidden_body(fm_ref, w1lo_ref, w1hi_ref, b1s_ref, out_ref):
    for j in range(MBA):
        fm = fm_ref[j]
        fa = fm[:, :HP]
        fb = fm[:, HP:]
        lo = lax.dot_general(fa, w1lo_ref[...], (((0,), (0,)), ((), ())),
                             preferred_element_type=jnp.float32)
        hi = lax.dot_general(fb, w1hi_ref[...], (((0,), (0,)), ((), ())),
                             preferred_element_type=jnp.float32)
        out_ref[j] = jnp.maximum(lo + hi + b1s_ref[...], 0.0)


@functools.lru_cache(maxsize=1)
def _make_gather():
    mesh = plsc.VectorSubcoreMesh(core_axis_name="c", subcore_axis_name="s",
                                  num_cores=_NC, num_subcores=_NS)

    @functools.partial(
        pl.kernel, mesh=mesh,
        out_type=jax.ShapeDtypeStruct((BS * K, 2 * NODE_DIM), jnp.float32),
        scratch_types=[
            pltpu.VMEM((K,), jnp.float32),
            pltpu.VMEM((K,), jnp.float32),
            pltpu.VMEM((K,), jnp.float32),
            pltpu.VMEM((K,), jnp.int32),
            pltpu.VMEM((K, 2 * NODE_DIM), jnp.float32),
            pltpu.SemaphoreType.DMA,
        ],
    )
    def gather_k(pp_hbm, table_hbm, out_hbm, v_v, r_v, c_v, idx_v, rows_v,
                 sem):
        w = lax.axis_index("s") * _NC + lax.axis_index("c")
        base = w * K
        pltpu.sync_copy(pp_hbm.at[pl.ds(base, K)], v_v)
        pltpu.sync_copy(pp_hbm.at[pl.ds(BS * K + base, K)], r_v)
        pltpu.sync_copy(pp_hbm.at[pl.ds(2 * BS * K + base, K)], c_v)
        for i in range(K // 16):
            s = pl.ds(i * 16, 16)
            vi = jnp.clip(v_v[s], 0.0, float(NUM_IMG - 1)).astype(jnp.int32)
            ri = jnp.clip(r_v[s], 0.0, float(H - 1)).astype(jnp.int32)
            ci = jnp.clip(c_v[s], 0.0, float(W - 1)).astype(jnp.int32)
            t = vi * (H * W) + ri * W + ci
            idx_v[s] = w * HP + jnp.bitwise_and(t, HP - 1)
        pltpu.async_copy(table_hbm.at[idx_v], rows_v, sem).wait()
        pltpu.sync_copy(rows_v, out_hbm.at[pl.ds(base, K)])

    return gather_k


def _out_body(g_ref, vcol_ref, w2_ref, b2_ref, out_ref):
    n = MBC * K
    vi = jnp.clip(vcol_ref[...].astype(jnp.int32), 0, NUM_IMG - 1)
    sel = jnp.where(vi >= NUM_IMG // 2, 1, 0)
    g = g_ref[...]
    col = lax.broadcasted_iota(jnp.int32, (n, 2 * NODE_DIM), 1)
    half = jnp.where(col < NODE_DIM, 0, 1)
    gm = jnp.where(half == sel, g, 0.0)
    w2s = jnp.concatenate([w2_ref[...], w2_ref[...]], axis=0)
    emb = lax.dot_general(gm, w2s, (((1,), (0,)), ((), ())),
                          preferred_element_type=jnp.float32) + b2_ref[...]
    norm = jnp.maximum(jnp.sqrt(jnp.sum(emb * emb, axis=-1, keepdims=True)),
                       1e-12)
    out_ref[...] = emb / norm


def kernel(feature_map, peak_positions, W1, b1, W2, b2):
    fm = feature_map.reshape(BS, C, P)
    zeros = jnp.zeros_like(W1)
    w1lo = jnp.concatenate([W1, zeros], axis=1)
    w1hi = jnp.concatenate([zeros, W1], axis=1)
    b1s = jnp.concatenate([b1, b1]).reshape(1, 2 * NODE_DIM)
    b2r = b2.reshape(1, NODE_DIM)

    table = pl.pallas_call(
        _hidden_body,
        grid=(BS // MBA,),
        in_specs=[
            pl.BlockSpec((MBA, C, P), lambda b: (b, 0, 0)),
            pl.BlockSpec((FEAT_DIM, 2 * NODE_DIM), lambda b: (0, 0)),
            pl.BlockSpec((FEAT_DIM, 2 * NODE_DIM), lambda b: (0, 0)),
            pl.BlockSpec((1, 2 * NODE_DIM), lambda b: (0, 0)),
        ],
        out_specs=pl.BlockSpec((MBA, HP, 2 * NODE_DIM), lambda b: (b, 0, 0)),
        out_shape=jax.ShapeDtypeStruct((BS, HP, 2 * NODE_DIM), jnp.float32),
    )(fm, w1lo, w1hi, b1s)

    pp = jnp.transpose(peak_positions, (2, 0, 1)).reshape(3 * BS * K)
    gathered = _make_gather()(pp, table.reshape(BS * HP, 2 * NODE_DIM))

    vcol = peak_positions[:, :, 0].reshape(BS * K, 1)
    out = pl.pallas_call(
        _out_body,
        grid=(BS // MBC,),
        in_specs=[
            pl.BlockSpec((MBC * K, 2 * NODE_DIM), lambda b: (b, 0)),
            pl.BlockSpec((MBC * K, 1), lambda b: (b, 0)),
            pl.BlockSpec((NODE_DIM, NODE_DIM), lambda b: (0, 0)),
            pl.BlockSpec((1, NODE_DIM), lambda b: (0, 0)),
        ],
        out_specs=pl.BlockSpec((MBC * K, NODE_DIM), lambda b: (b, 0)),
        out_shape=jax.ShapeDtypeStruct((BS * K, NODE_DIM), jnp.float32),
    )(gathered, vcol, W2, b2r)
    return out.reshape(BS, K, NODE_DIM)
